# baseline (speedup 1.0000x reference)
import functools

import numpy as np
import jax
import jax.numpy as jnp
from jax import lax
from jax.experimental import pallas as pl
from jax.experimental.pallas import tpu as pltpu, tpu_sc as plsc

_D_MODEL = 1024
_MAX_LEN = 8192


def _pos_encoding_np(max_len, d_model):
    pos = np.arange(max_len, dtype=np.float32)[:, None]
    i = np.arange(0, d_model, 2, dtype=np.float32)
    div = np.power(10000.0, i / d_model)
    enc = np.zeros((max_len, d_model), dtype=np.float32)
    enc[:, 0::2] = np.sin(pos / div)
    enc[:, 1::2] = np.cos(pos / div)
    return enc


_PE_NP = _pos_encoding_np(_MAX_LEN, _D_MODEL)


@functools.lru_cache(maxsize=None)
def _build(B, S, D, C):
    info = plsc.get_sparse_core_info()
    NW = info.num_cores * info.num_subcores
    P = S // NW
    n_chunks = P // C
    v16 = D // 16

    mesh = plsc.VectorSubcoreMesh(core_axis_name="c", subcore_axis_name="s")

    NBUF = 6

    @functools.partial(
        pl.kernel,
        mesh=mesh,
        out_type=jax.ShapeDtypeStruct((B * S, D), jnp.float32),
        scratch_types=[
            pltpu.VMEM((B * P,), jnp.int32),
            pltpu.VMEM((NBUF, B * C, D), jnp.float32),
            pltpu.VMEM((NBUF, C, D), jnp.float32),
        ] + [pltpu.SemaphoreType.DMA] * (2 * NBUF),
    )
    def k(idx_hbm, table_hbm, pe_hbm, out_hbm, idx_v, rows_v, pe_v,
          sg0, sg1, sg2, sg3, sg4, sg5, so0, so1, so2, so3, so4, so5):
        sg = (sg0, sg1, sg2, sg3, sg4, sg5)
        so = (so0, so1, so2, so3, so4, so5)
        wid = lax.axis_index("s") * info.num_cores + lax.axis_index("c")
        p0 = wid * P
        pltpu.sync_copy(idx_hbm.at[pl.ds(wid * B * P, B * P)], idx_v)

        def issue_gathers(ci, buf):
            cps = [
                pltpu.async_copy(
                    table_hbm.at[idx_v.at[pl.ds(ci * B * C, B * C)]],
                    rows_v.at[buf],
                    sg[buf],
                ),
                pltpu.async_copy(pe_hbm.at[pl.ds(p0 + ci * C, C)],
                                 pe_v.at[buf], sg[buf]),
            ]
            return cps

        out_cps = {b: [] for b in range(NBUF)}
        gat_cps = {}
        for j in range(NBUF - 1):
            gat_cps[j] = issue_gathers(j, j)

        for ci in range(n_chunks):
            buf = ci % NBUF
            for cp in gat_cps[buf]:
                cp.wait()

            def add_body(r, _, buf=buf):
                def col_body(j, _):
                    col = j * 16
                    pe_vec = pe_v[buf, r, pl.ds(col, 16)]
                    for b in range(B):
                        plsc.addupdate(
                            rows_v.at[buf, b * C + r, pl.ds(col, 16)], pe_vec
                        )
                    return 0

                lax.fori_loop(0, v16, col_body, 0, unroll=4)
                return 0

            lax.fori_loop(0, C, add_body, 0)
            off = ci * C
            out_cps[buf] = [
                pltpu.async_copy(
                    rows_v.at[buf, pl.ds(b * C, C)],
                    out_hbm.at[pl.ds(b * S + p0 + off, C)],
                    so[buf],
                )
                for b in range(B)
            ]
            if ci + 1 < n_chunks:
                nb = (ci + NBUF - 1) % NBUF
                for cp in out_cps[nb]:
                    cp.wait()
                out_cps[nb] = []
                if ci + NBUF - 1 < n_chunks:
                    gat_cps[nb] = issue_gathers(ci + NBUF - 1, nb)
        for b in range(NBUF):
            for cp in out_cps[b]:
                cp.wait()

    return k


def kernel(x, tok_table):
    B, S = x.shape
    V, D = tok_table.shape
    C = 4
    NW = 32
    n_chunks = S // NW // C
    idx = (
        x.astype(jnp.int32)
        .reshape(B, NW, n_chunks, C)
        .transpose(1, 2, 0, 3)
        .reshape(-1)
    )
    pe = jnp.asarray(_PE_NP[:S], dtype=jnp.float32)
    out = _build(B, S, D, C)(idx, tok_table, pe)
    return out.reshape(B, S, D)


# speedup vs baseline: 1.0308x; 1.0308x over previous
import functools

import numpy as np
import jax
import jax.numpy as jnp
from jax import lax
from jax.experimental import pallas as pl
from jax.experimental.pallas import tpu as pltpu, tpu_sc as plsc

_D_MODEL = 1024
_MAX_LEN = 8192


def _pos_encoding_np(max_len, d_model):
    pos = np.arange(max_len, dtype=np.float32)[:, None]
    i = np.arange(0, d_model, 2, dtype=np.float32)
    div = np.power(10000.0, i / d_model)
    enc = np.zeros((max_len, d_model), dtype=np.float32)
    enc[:, 0::2] = np.sin(pos / div)
    enc[:, 1::2] = np.cos(pos / div)
    return enc


_PE_NP = _pos_encoding_np(_MAX_LEN, _D_MODEL)


@functools.lru_cache(maxsize=None)
def _build(B, S, D, C):
    info = plsc.get_sparse_core_info()
    NW = info.num_cores * info.num_subcores
    P = S // NW
    n_chunks = P // C
    v16 = D // 16

    mesh = plsc.VectorSubcoreMesh(core_axis_name="c", subcore_axis_name="s")

    NBUF = 3
    H = C // 2

    @functools.partial(
        pl.kernel,
        mesh=mesh,
        out_type=jax.ShapeDtypeStruct((B * S, D), jnp.float32),
        scratch_types=[
            pltpu.VMEM((B * P,), jnp.int32),
            pltpu.VMEM((NBUF, B * C, D), jnp.float32),
            pltpu.VMEM((NBUF, C, D), jnp.float32),
        ] + [pltpu.SemaphoreType.DMA] * (2 * NBUF),
    )
    def k(idx_hbm, table_hbm, pe_hbm, out_hbm, idx_v, rows_v, pe_v,
          sg0, sg1, sg2, so0, so1, so2):
        sg = (sg0, sg1, sg2)
        so = (so0, so1, so2)
        wid = lax.axis_index("s") * info.num_cores + lax.axis_index("c")
        p0 = wid * P
        for b in range(B):
            pltpu.sync_copy(
                idx_hbm.at[pl.ds(b * S + p0, P)], idx_v.at[pl.ds(b * P, P)]
            )

        def issue_gathers(ci, buf):
            cps = [
                pltpu.async_copy(
                    table_hbm.at[idx_v.at[pl.ds(b * P + ci * C, C)]],
                    rows_v.at[buf, pl.ds(b * C, C)],
                    sg[buf],
                )
                for b in range(B)
            ]
            cps.append(
                pltpu.async_copy(pe_hbm.at[pl.ds(p0 + ci * C, C)],
                                 pe_v.at[buf], sg[buf])
            )
            return cps

        def add_rows(buf, r0):
            def add_body(r, _):
                def col_body(j, _):
                    col = j * 16
                    pe_vec = pe_v[buf, r, pl.ds(col, 16)]
                    for b in range(B):
                        plsc.addupdate(
                            rows_v.at[buf, b * C + r, pl.ds(col, 16)], pe_vec
                        )
                    return 0

                lax.fori_loop(0, v16, col_body, 0, unroll=4)
                return 0

            lax.fori_loop(r0, r0 + H, add_body, 0)

        out_cps = {b: [] for b in range(NBUF)}
        gat_cps = {}
        for j in range(NBUF - 1):
            gat_cps[j] = issue_gathers(j, j)

        for ci in range(n_chunks):
            buf = ci % NBUF
            off = ci * C
            for cp in gat_cps[buf]:
                cp.wait()
            add_rows(buf, 0)
            cps = [
                pltpu.async_copy(
                    rows_v.at[buf, pl.ds(b * C, H)],
                    out_hbm.at[pl.ds(b * S + p0 + off, H)],
                    so[buf],
                )
                for b in range(B)
            ]
            add_rows(buf, H)
            cps.extend(
                pltpu.async_copy(
                    rows_v.at[buf, pl.ds(b * C + H, C - H)],
                    out_hbm.at[pl.ds(b * S + p0 + off + H, C - H)],
                    so[buf],
                )
                for b in range(B)
            )
            out_cps[buf] = cps
            if ci + 1 < n_chunks:
                nb = (ci + NBUF - 1) % NBUF
                for cp in out_cps[nb]:
                    cp.wait()
                out_cps[nb] = []
                if ci + NBUF - 1 < n_chunks:
                    gat_cps[nb] = issue_gathers(ci + NBUF - 1, nb)
        for b in range(NBUF):
            for cp in out_cps[b]:
                cp.wait()

    return k


def kernel(x, tok_table):
    B, S = x.shape
    V, D = tok_table.shape
    C = 8
    idx = x.reshape(-1).astype(jnp.int32)
    pe = jnp.asarray(_PE_NP[:S], dtype=jnp.float32)
    out = _build(B, S, D, C)(idx, tok_table, pe)
    return out.reshape(B, S, D)


# speedup vs baseline: 1.0699x; 1.0380x over previous
import functools

import numpy as np
import jax
import jax.numpy as jnp
from jax import lax
from jax.experimental import pallas as pl
from jax.experimental.pallas import tpu as pltpu, tpu_sc as plsc

_D_MODEL = 1024
_MAX_LEN = 8192


def _pos_encoding_np(max_len, d_model):
    pos = np.arange(max_len, dtype=np.float32)[:, None]
    i = np.arange(0, d_model, 2, dtype=np.float32)
    div = np.power(10000.0, i / d_model)
    enc = np.zeros((max_len, d_model), dtype=np.float32)
    enc[:, 0::2] = np.sin(pos / div)
    enc[:, 1::2] = np.cos(pos / div)
    return enc


_PE_NP = _pos_encoding_np(_MAX_LEN, _D_MODEL)


@functools.lru_cache(maxsize=None)
def _build(B, S, D, C):
    info = plsc.get_sparse_core_info()
    NW = info.num_cores * info.num_subcores
    P = S // NW
    n_chunks = P // C
    v16 = D // 16

    mesh = plsc.VectorSubcoreMesh(core_axis_name="c", subcore_axis_name="s")

    NBUF = 3
    H = C // 2

    @functools.partial(
        pl.kernel,
        mesh=mesh,
        out_type=jax.ShapeDtypeStruct((B * S, D), jnp.float32),
        scratch_types=[
            pltpu.VMEM((B * P,), jnp.int32),
            pltpu.VMEM((NBUF, B * C, D), jnp.float32),
            pltpu.VMEM((NBUF, C, D), jnp.float32),
        ] + [pltpu.SemaphoreType.DMA] * (2 * NBUF),
    )
    def k(idx_hbm, table_hbm, pe_hbm, out_hbm, idx_v, rows_v, pe_v,
          sg0, sg1, sg2, so0, so1, so2):
        sg = (sg0, sg1, sg2)
        so = (so0, so1, so2)
        wid = lax.axis_index("s") * info.num_cores + lax.axis_index("c")
        p0 = wid * P
        for b in range(B):
            pltpu.sync_copy(
                idx_hbm.at[pl.ds(b * S + p0, P)], idx_v.at[pl.ds(b * P, P)]
            )

        def issue_gathers(ci, buf):
            cps = [
                pltpu.async_copy(
                    table_hbm.at[idx_v.at[pl.ds(b * P + ci * C, C)]],
                    rows_v.at[buf, pl.ds(b * C, C)],
                    sg[buf],
                )
                for b in range(B)
            ]
            cps.append(
                pltpu.async_copy(pe_hbm.at[pl.ds(p0 + ci * C, C)],
                                 pe_v.at[buf], sg[buf])
            )
            return cps

        def add_rows(buf, r0, rn):
            def add_body(r, _):
                def col_body(j, _):
                    col = j * 16
                    pe_vec = pe_v[buf, r, pl.ds(col, 16)]
                    for b in range(B):
                        plsc.addupdate(
                            rows_v.at[buf, b * C + r, pl.ds(col, 16)], pe_vec
                        )
                    return 0

                lax.fori_loop(0, v16, col_body, 0, unroll=4)
                return 0

            lax.fori_loop(r0, rn, add_body, 0)

        out_cps = {b: [] for b in range(NBUF)}
        gat_cps = {}
        for j in range(NBUF - 1):
            gat_cps[j] = issue_gathers(j, j)

        for ci in range(n_chunks):
            buf = ci % NBUF
            off = ci * C
            for cp in gat_cps[buf]:
                cp.wait()
            add_rows(buf, 0, C)
            out_cps[buf] = [
                pltpu.async_copy(
                    rows_v.at[buf, pl.ds(b * C, C)],
                    out_hbm.at[pl.ds(b * S + p0 + off, C)],
                    so[buf],
                )
                for b in range(B)
            ]
            if ci + 1 < n_chunks:
                nb = (ci + NBUF - 1) % NBUF
                for cp in out_cps[nb]:
                    cp.wait()
                out_cps[nb] = []
                if ci + NBUF - 1 < n_chunks:
                    gat_cps[nb] = issue_gathers(ci + NBUF - 1, nb)
        for b in range(NBUF):
            for cp in out_cps[b]:
                cp.wait()

    return k


def kernel(x, tok_table):
    B, S = x.shape
    V, D = tok_table.shape
    C = 8
    idx = x.reshape(-1).astype(jnp.int32)
    pe = jnp.asarray(_PE_NP[:S], dtype=jnp.float32)
    out = _build(B, S, D, C)(idx, tok_table, pe)
    return out.reshape(B, S, D)


# speedup vs baseline: 1.1270x; 1.0534x over previous
import functools

import numpy as np
import jax
import jax.numpy as jnp
from jax import lax
from jax.experimental import pallas as pl
from jax.experimental.pallas import tpu as pltpu, tpu_sc as plsc

_D_MODEL = 1024
_MAX_LEN = 8192


def _pos_encoding_np(max_len, d_model):
    pos = np.arange(max_len, dtype=np.float32)[:, None]
    i = np.arange(0, d_model, 2, dtype=np.float32)
    div = np.power(10000.0, i / d_model)
    enc = np.zeros((max_len, d_model), dtype=np.float32)
    enc[:, 0::2] = np.sin(pos / div)
    enc[:, 1::2] = np.cos(pos / div)
    return enc


_PE_NP = _pos_encoding_np(_MAX_LEN, _D_MODEL)


@functools.lru_cache(maxsize=None)
def _build(B, S, D, C):
    info = plsc.get_sparse_core_info()
    NW = info.num_cores * info.num_subcores
    P = S // NW
    n_chunks = P // C
    v16 = D // 16

    mesh = plsc.VectorSubcoreMesh(core_axis_name="c", subcore_axis_name="s")

    NBUF = 3
    H = C // 2

    @functools.partial(
        pl.kernel,
        mesh=mesh,
        out_type=jax.ShapeDtypeStruct((B * S, D), jnp.float32),
        scratch_types=[
            pltpu.VMEM((B * P,), jnp.int32),
            pltpu.VMEM((NBUF, B * C, D), jnp.float32),
            pltpu.VMEM((NBUF, C, D), jnp.float32),
        ] + [pltpu.SemaphoreType.DMA] * (2 * NBUF),
    )
    def k(idx_hbm, table_hbm, pe_hbm, out_hbm, idx_v, rows_v, pe_v,
          sg0, sg1, sg2, so0, so1, so2):
        sg = (sg0, sg1, sg2)
        so = (so0, so1, so2)
        wid = lax.axis_index("s") * info.num_cores + lax.axis_index("c")
        p0 = wid * P
        idx_cps = [
            pltpu.async_copy(
                idx_hbm.at[pl.ds(b * S + p0, P)], idx_v.at[pl.ds(b * P, P)],
                so0,
            )
            for b in range(B)
        ]
        for cp in idx_cps:
            cp.wait()

        def issue_gathers(ci, buf):
            cps = [
                pltpu.async_copy(
                    table_hbm.at[idx_v.at[pl.ds(b * P + ci * C, C)]],
                    rows_v.at[buf, pl.ds(b * C, C)],
                    sg[buf],
                )
                for b in range(B)
            ]
            cps.append(
                pltpu.async_copy(pe_hbm.at[pl.ds(p0 + ci * C, C)],
                                 pe_v.at[buf], sg[buf])
            )
            return cps

        def add_rows(buf, r0, rn):
            def add_body(r, _):
                def col_body(j, _):
                    col = j * 16
                    pe_vec = pe_v[buf, r, pl.ds(col, 16)]
                    for b in range(B):
                        plsc.addupdate(
                            rows_v.at[buf, b * C + r, pl.ds(col, 16)], pe_vec
                        )
                    return 0

                lax.fori_loop(0, v16, col_body, 0, unroll=8)
                return 0

            lax.fori_loop(r0, rn, add_body, 0)

        out_cps = {b: [] for b in range(NBUF)}
        gat_cps = {}
        for j in range(NBUF - 1):
            gat_cps[j] = issue_gathers(j, j)

        for ci in range(n_chunks):
            buf = ci % NBUF
            off = ci * C
            for cp in gat_cps[buf]:
                cp.wait()
            add_rows(buf, 0, C)
            out_cps[buf] = [
                pltpu.async_copy(
                    rows_v.at[buf, pl.ds(b * C, C)],
                    out_hbm.at[pl.ds(b * S + p0 + off, C)],
                    so[buf],
                )
                for b in range(B)
            ]
            if ci + 1 < n_chunks:
                nb = (ci + NBUF - 1) % NBUF
                for cp in out_cps[nb]:
                    cp.wait()
                out_cps[nb] = []
                if ci + NBUF - 1 < n_chunks:
                    gat_cps[nb] = issue_gathers(ci + NBUF - 1, nb)
        for b in range(NBUF):
            for cp in out_cps[b]:
                cp.wait()

    return k


def kernel(x, tok_table):
    B, S = x.shape
    V, D = tok_table.shape
    C = 8
    idx = x.reshape(-1).astype(jnp.int32)
    pe = jnp.asarray(_PE_NP[:S], dtype=jnp.float32)
    out = _build(B, S, D, C)(idx, tok_table, pe)
    return out.reshape(B, S, D)
